# baseline (speedup 1.0000x reference)
import functools

import jax
import jax.numpy as jnp
from jax import lax
from jax.experimental import pallas as pl
from jax.experimental.pallas import tpu as pltpu
from jax.experimental.pallas import tpu_sc as plsc

NUM_HEADS = 12
AREA = 576 * 576
TABLE_N = 2209 * NUM_HEADS
NC, NS, L = 2, 16, 16
NW = NC * NS
CHUNK = AREA // NW
BATCH = 16
HG = 6


def _sc_gather_bias6(table_flat, idx_flat, h0):
    mesh = plsc.VectorSubcoreMesh(core_axis_name="c", subcore_axis_name="s")

    @functools.partial(
        pl.kernel,
        out_type=jax.ShapeDtypeStruct((HG, AREA), jnp.float32),
        mesh=mesh,
        scratch_types=[
            pltpu.VMEM((TABLE_N,), jnp.float32),
            pltpu.VMEM((CHUNK,), jnp.int32),
            pltpu.VMEM((HG, CHUNK), jnp.float32),
            pltpu.SemaphoreType.DMA,
        ],
        compiler_params=pltpu.CompilerParams(
            needs_layout_passes=False, use_tc_tiling_on_sc=False
        ),
        name=f"bias_gather_h{h0}",
    )
    def k(table_hbm, idx_hbm, out_hbm, tab_v, idx_v, rows_v, sem):
        wid = lax.axis_index("s") * NC + lax.axis_index("c")
        base = wid * CHUNK
        tab_cp = pltpu.async_copy(table_hbm, tab_v, sem)
        idx_cp = pltpu.async_copy(idx_hbm.at[pl.ds(base, CHUNK)], idx_v, sem)
        tab_cp.wait()
        idx_cp.wait()

        def body(i, c):
            iv = idx_v[pl.ds(i * L, L)]
            g0 = iv * NUM_HEADS + h0
            for h in range(HG):
                rows_v[h, pl.ds(i * L, L)] = plsc.load_gather(tab_v, [g0 + h])
            return c

        lax.fori_loop(0, CHUNK // L, body, 0, unroll=2)
        cps = [
            pltpu.async_copy(rows_v.at[h], out_hbm.at[h, pl.ds(base, CHUNK)], sem)
            for h in range(HG)
        ]
        for cp in cps:
            cp.wait()

    return k(table_flat, idx_flat)


def _add_body(attn_ref, bias_ref, out_ref):
    out_ref[...] = attn_ref[...] + bias_ref[...]


def _add_body_alias(attn_ref, bias_ref, _, out_ref):
    out_ref[...] = attn_ref[...] + bias_ref[...]


def _tc_add_group0(attn, bias3):
    return pl.pallas_call(
        _add_body,
        grid=(BATCH,),
        in_specs=[
            pl.BlockSpec((1, HG, 576, 576), lambda b: (b, 0, 0, 0)),
            pl.BlockSpec((HG, 576, 576), lambda b: (0, 0, 0)),
        ],
        out_specs=pl.BlockSpec((1, HG, 576, 576), lambda b: (b, 0, 0, 0)),
        out_shape=jax.ShapeDtypeStruct(attn.shape, attn.dtype),
    )(attn, bias3)


def _tc_add_group1(attn, bias3, partial_out):
    return pl.pallas_call(
        _add_body_alias,
        grid=(BATCH,),
        in_specs=[
            pl.BlockSpec((1, HG, 576, 576), lambda b: (b, 1, 0, 0)),
            pl.BlockSpec((HG, 576, 576), lambda b: (0, 0, 0)),
            pl.BlockSpec(memory_space=pl.ANY),
        ],
        out_specs=pl.BlockSpec((1, HG, 576, 576), lambda b: (b, 1, 0, 0)),
        out_shape=jax.ShapeDtypeStruct(attn.shape, attn.dtype),
        input_output_aliases={2: 0},
    )(attn, bias3, partial_out)


def kernel(attn, rel_pos_bias_table, rel_pos_index):
    table_flat = rel_pos_bias_table.reshape(TABLE_N)
    idx_flat = rel_pos_index.reshape(AREA).astype(jnp.int32)
    bias0 = _sc_gather_bias6(table_flat, idx_flat, 0).reshape(HG, 576, 576)
    bias1 = _sc_gather_bias6(table_flat, idx_flat, HG).reshape(HG, 576, 576)
    out = _tc_add_group0(attn, bias0)
    return _tc_add_group1(attn, bias1, out)


# speedup vs baseline: 1.1583x; 1.1583x over previous
import functools

import jax
import jax.numpy as jnp
from jax import lax
from jax.experimental import pallas as pl
from jax.experimental.pallas import tpu as pltpu
from jax.experimental.pallas import tpu_sc as plsc

NUM_HEADS = 12
AREA = 576 * 576
TABLE_N = 2209 * NUM_HEADS
NC, NS, L = 2, 16, 16
NW = NC * NS
CHUNK = AREA // NW
BATCH = 16
HG = 6


def _sc_gather_bias(table_flat, idx_flat):
    mesh = plsc.VectorSubcoreMesh(core_axis_name="c", subcore_axis_name="s")
    HALF = CHUNK // 2

    @functools.partial(
        pl.kernel,
        out_type=jax.ShapeDtypeStruct((NUM_HEADS, AREA), jnp.float32),
        mesh=mesh,
        scratch_types=[
            pltpu.VMEM((TABLE_N,), jnp.float32),
            pltpu.VMEM((CHUNK,), jnp.int32),
            pltpu.VMEM((NUM_HEADS, HALF), jnp.float32),
            pltpu.SemaphoreType.DMA,
        ],
        compiler_params=pltpu.CompilerParams(
            needs_layout_passes=False, use_tc_tiling_on_sc=False
        ),
        name="bias_gather",
    )
    def k(table_hbm, idx_hbm, out_hbm, tab_v, idx_v, rows_v, sem):
        wid = lax.axis_index("s") * NC + lax.axis_index("c")
        base = wid * CHUNK
        tab_cp = pltpu.async_copy(table_hbm, tab_v, sem)
        idx_cp = pltpu.async_copy(idx_hbm.at[pl.ds(base, CHUNK)], idx_v, sem)
        tab_cp.wait()
        idx_cp.wait()

        def half_body(half):
            off = half * HALF

            @functools.partial(
                plsc.parallel_loop, 0, HALF // L, unroll=4
            )
            def body(i):
                iv = idx_v[pl.ds(off + i * L, L)]
                g0 = iv * NUM_HEADS
                for h in range(NUM_HEADS):
                    rows_v[h, pl.ds(i * L, L)] = plsc.load_gather(tab_v, [g0 + h])

            cps = [
                pltpu.async_copy(
                    rows_v.at[h], out_hbm.at[h, pl.ds(base + off, HALF)], sem
                )
                for h in range(NUM_HEADS)
            ]
            for cp in cps:
                cp.wait()

        half_body(0)
        half_body(1)

    return k(table_flat, idx_flat)


def _tc_add(attn, bias3):
    def body(attn_ref, bias_ref, out_ref):
        out_ref[...] = attn_ref[...] + bias_ref[...]

    return pl.pallas_call(
        body,
        grid=(NUM_HEADS // HG, BATCH),
        in_specs=[
            pl.BlockSpec((1, HG, 576, 576), lambda h, b: (b, h, 0, 0)),
            pl.BlockSpec((HG, 576, 576), lambda h, b: (h, 0, 0)),
        ],
        out_specs=pl.BlockSpec((1, HG, 576, 576), lambda h, b: (b, h, 0, 0)),
        out_shape=jax.ShapeDtypeStruct(attn.shape, attn.dtype),
    )(attn, bias3)


def kernel(attn, rel_pos_bias_table, rel_pos_index):
    table_flat = rel_pos_bias_table.reshape(TABLE_N)
    idx_flat = rel_pos_index.reshape(AREA).astype(jnp.int32)
    bias = _sc_gather_bias(table_flat, idx_flat)
    bias3 = bias.reshape(NUM_HEADS, 576, 576)
    return _tc_add(attn, bias3)
